# baseline (speedup 1.0000x reference)
import functools

import jax
import jax.numpy as jnp
from jax.experimental import pallas as pl
from jax.experimental.pallas import tpu as pltpu


def _posenc_kernel(x_ref, row_ref, col_ref, o_ref, pos_ref, *, H, W, CH):
    i = pl.program_id(0)
    HW = H * W

    @pl.when(i == 0)
    def _build_pos():
        r = jax.lax.broadcasted_iota(jnp.int32, (HW, H), 0)
        k = jax.lax.broadcasted_iota(jnp.int32, (HW, H), 1)
        Eh = (r // W == k).astype(jnp.float32)
        Ew = (r % W == k).astype(jnp.float32)
        pos_ref[:, :CH] = jax.lax.dot(
            Eh, row_ref[:H, :], precision=jax.lax.Precision.HIGHEST,
            preferred_element_type=jnp.float32)
        pos_ref[:, CH:] = jax.lax.dot(
            Ew, col_ref[:W, :], precision=jax.lax.Precision.HIGHEST,
            preferred_element_type=jnp.float32)

    o_ref[...] = x_ref[...] + pos_ref[...][None]


def kernel(x, row_embed, col_embed):
    b, c, h, w = x.shape
    ch = c // 2
    hw = h * w
    G = 32
    xt = jnp.transpose(x, (0, 2, 3, 1)).reshape(b, hw, c)
    body = functools.partial(_posenc_kernel, H=h, W=w, CH=ch)
    out = pl.pallas_call(
        body,
        grid=(b // G,),
        in_specs=[
            pl.BlockSpec((G, hw, c), lambda i: (i, 0, 0)),
            pl.BlockSpec(row_embed.shape, lambda i: (0, 0)),
            pl.BlockSpec(col_embed.shape, lambda i: (0, 0)),
        ],
        out_specs=pl.BlockSpec((G, hw, c), lambda i: (i, 0, 0)),
        out_shape=jax.ShapeDtypeStruct((b, hw, c), x.dtype),
        scratch_shapes=[pltpu.VMEM((hw, c), jnp.float32)],
    )(xt, row_embed, col_embed)
    return out.reshape(b, h, w, c).transpose(0, 3, 1, 2)


# speedup vs baseline: 1.1636x; 1.1636x over previous
import functools

import jax
import jax.numpy as jnp
from jax.experimental import pallas as pl
from jax.experimental.pallas import tpu as pltpu


def _posenc_kernel(x_ref, row_ref, col_ref, o_ref,
                   in_bufs, out_bufs, pos_ref, in_sems, out_sems,
                   *, H, W, CH, R, K, N):
    HW = H * W
    T = R // HW

    def in_copy(idx, slot):
        return pltpu.make_async_copy(
            x_ref.at[pl.ds(idx * R, R)], in_bufs.at[slot], in_sems.at[slot])

    def out_copy(idx, slot):
        return pltpu.make_async_copy(
            out_bufs.at[slot], o_ref.at[pl.ds(idx * R, R)], out_sems.at[slot])

    for k in range(K):
        in_copy(k, k).start()

    r = jax.lax.broadcasted_iota(jnp.int32, (HW, H), 0)
    k = jax.lax.broadcasted_iota(jnp.int32, (HW, H), 1)
    Eh = (r // W == k).astype(jnp.float32)
    Ew = (r % W == k).astype(jnp.float32)
    pos_ref[:, :CH] = jax.lax.dot(
        Eh, row_ref[:H, :], precision=jax.lax.Precision.HIGHEST,
        preferred_element_type=jnp.float32)
    pos_ref[:, CH:] = jax.lax.dot(
        Ew, col_ref[:W, :], precision=jax.lax.Precision.HIGHEST,
        preferred_element_type=jnp.float32)

    def body(i, _):
        slot = jax.lax.rem(i, K)
        in_copy(i, slot).wait()

        @pl.when(i >= K)
        def _():
            out_copy(i - K, slot).wait()

        for t in range(T):
            sl = pl.ds(t * HW, HW)
            out_bufs[slot, sl, :] = in_bufs[slot, sl, :] + pos_ref[...]
        out_copy(i, slot).start()

        @pl.when(i + K < N)
        def _():
            in_copy(i + K, slot).start()

        return 0

    jax.lax.fori_loop(0, N, body, 0)

    def drain(i, _):
        slot = jax.lax.rem(i, K)
        out_copy(i, slot).wait()
        return 0

    jax.lax.fori_loop(N - K, N, drain, 0)


def kernel(x, row_embed, col_embed):
    b, c, h, w = x.shape
    ch = c // 2
    hw = h * w
    T = 2
    K = 4
    R = T * hw
    N = b // T
    xt = jnp.transpose(x, (0, 2, 3, 1)).reshape(b * hw, c)
    body = functools.partial(_posenc_kernel, H=h, W=w, CH=ch, R=R, K=K, N=N)
    out = pl.pallas_call(
        body,
        in_specs=[
            pl.BlockSpec(memory_space=pltpu.HBM),
            pl.BlockSpec(memory_space=pltpu.VMEM),
            pl.BlockSpec(memory_space=pltpu.VMEM),
        ],
        out_specs=pl.BlockSpec(memory_space=pltpu.HBM),
        out_shape=jax.ShapeDtypeStruct((b * hw, c), x.dtype),
        scratch_shapes=[
            pltpu.VMEM((K, R, c), jnp.float32),
            pltpu.VMEM((K, R, c), jnp.float32),
            pltpu.VMEM((hw, c), jnp.float32),
            pltpu.SemaphoreType.DMA((K,)),
            pltpu.SemaphoreType.DMA((K,)),
        ],
    )(xt, row_embed, col_embed)
    return out.reshape(b, h, w, c).transpose(0, 3, 1, 2)


# speedup vs baseline: 1.1768x; 1.0113x over previous
import functools

import jax
import jax.numpy as jnp
from jax.experimental import pallas as pl
from jax.experimental.pallas import tpu as pltpu


def _posenc_kernel(x_ref, row_ref, col_ref, o_ref,
                   in_bufs, out_bufs, pos_ref, in_sems, out_sems,
                   *, H, W, CH, R, K, N):
    HW = H * W
    T = R // HW

    def in_copy(idx, slot):
        return pltpu.make_async_copy(
            x_ref.at[pl.ds(idx * R, R)], in_bufs.at[slot], in_sems.at[slot])

    def out_copy(idx, slot):
        return pltpu.make_async_copy(
            out_bufs.at[slot], o_ref.at[pl.ds(idx * R, R)], out_sems.at[slot])

    for k in range(K):
        in_copy(k, k).start()

    r = jax.lax.broadcasted_iota(jnp.int32, (HW, H), 0)
    k = jax.lax.broadcasted_iota(jnp.int32, (HW, H), 1)
    Eh = (r // W == k).astype(jnp.float32)
    Ew = (r % W == k).astype(jnp.float32)
    pos_ref[:, :CH] = jax.lax.dot(
        Eh, row_ref[:H, :], precision=jax.lax.Precision.HIGHEST,
        preferred_element_type=jnp.float32)
    pos_ref[:, CH:] = jax.lax.dot(
        Ew, col_ref[:W, :], precision=jax.lax.Precision.HIGHEST,
        preferred_element_type=jnp.float32)

    def body(i, _):
        slot = jax.lax.rem(i, K)
        in_copy(i, slot).wait()

        @pl.when(i >= K)
        def _():
            out_copy(i - K, slot).wait()

        for t in range(T):
            sl = pl.ds(t * HW, HW)
            out_bufs[slot, sl, :] = in_bufs[slot, sl, :] + pos_ref[...]
        out_copy(i, slot).start()

        @pl.when(i + K < N)
        def _():
            in_copy(i + K, slot).start()

        return 0

    jax.lax.fori_loop(0, N, body, 0)

    def drain(i, _):
        slot = jax.lax.rem(i, K)
        out_copy(i, slot).wait()
        return 0

    jax.lax.fori_loop(N - K, N, drain, 0)


def kernel(x, row_embed, col_embed):
    b, c, h, w = x.shape
    ch = c // 2
    hw = h * w
    T = 4
    K = 4
    R = T * hw
    N = b // T
    xt = jnp.transpose(x, (0, 2, 3, 1)).reshape(b * hw, c)
    body = functools.partial(_posenc_kernel, H=h, W=w, CH=ch, R=R, K=K, N=N)
    out = pl.pallas_call(
        body,
        in_specs=[
            pl.BlockSpec(memory_space=pltpu.HBM),
            pl.BlockSpec(memory_space=pltpu.VMEM),
            pl.BlockSpec(memory_space=pltpu.VMEM),
        ],
        out_specs=pl.BlockSpec(memory_space=pltpu.HBM),
        out_shape=jax.ShapeDtypeStruct((b * hw, c), x.dtype),
        scratch_shapes=[
            pltpu.VMEM((K, R, c), jnp.float32),
            pltpu.VMEM((K, R, c), jnp.float32),
            pltpu.VMEM((hw, c), jnp.float32),
            pltpu.SemaphoreType.DMA((K,)),
            pltpu.SemaphoreType.DMA((K,)),
        ],
    )(xt, row_embed, col_embed)
    return out.reshape(b, h, w, c).transpose(0, 3, 1, 2)


# speedup vs baseline: 1.1916x; 1.0126x over previous
import functools

import jax
import jax.numpy as jnp
from jax.experimental import pallas as pl
from jax.experimental.pallas import tpu as pltpu


def _posenc_kernel(x_ref, row_ref, col_ref, o_ref,
                   in_bufs, out_bufs, pos_ref, in_sems, out_sems,
                   *, H, W, CH, R, K, N):
    HW = H * W
    T = R // HW

    def in_copy(idx, slot):
        return pltpu.make_async_copy(
            x_ref.at[pl.ds(idx * R, R)], in_bufs.at[slot], in_sems.at[slot])

    def out_copy(idx, slot):
        return pltpu.make_async_copy(
            out_bufs.at[slot], o_ref.at[pl.ds(idx * R, R)], out_sems.at[slot])

    for k in range(K):
        in_copy(k, k).start()

    r = jax.lax.broadcasted_iota(jnp.int32, (HW, H), 0)
    k = jax.lax.broadcasted_iota(jnp.int32, (HW, H), 1)
    Eh = (r // W == k).astype(jnp.float32)
    Ew = (r % W == k).astype(jnp.float32)
    pos_ref[:, :CH] = jax.lax.dot(
        Eh, row_ref[:H, :], precision=jax.lax.Precision.HIGHEST,
        preferred_element_type=jnp.float32)
    pos_ref[:, CH:] = jax.lax.dot(
        Ew, col_ref[:W, :], precision=jax.lax.Precision.HIGHEST,
        preferred_element_type=jnp.float32)

    def body(i, _):
        slot = jax.lax.rem(i, K)
        in_copy(i, slot).wait()

        @pl.when(i >= K)
        def _():
            out_copy(i - K, slot).wait()

        for t in range(T):
            sl = pl.ds(t * HW, HW)
            out_bufs[slot, sl, :] = in_bufs[slot, sl, :] + pos_ref[...]
        out_copy(i, slot).start()

        @pl.when(i + K < N)
        def _():
            in_copy(i + K, slot).start()

        return 0

    jax.lax.fori_loop(0, N, body, 0)

    def drain(i, _):
        slot = jax.lax.rem(i, K)
        out_copy(i, slot).wait()
        return 0

    jax.lax.fori_loop(N - K, N, drain, 0)


def kernel(x, row_embed, col_embed):
    b, c, h, w = x.shape
    ch = c // 2
    hw = h * w
    T = 8
    K = 3
    R = T * hw
    N = b // T
    xt = jnp.transpose(x, (0, 2, 3, 1)).reshape(b * hw, c)
    body = functools.partial(_posenc_kernel, H=h, W=w, CH=ch, R=R, K=K, N=N)
    out = pl.pallas_call(
        body,
        in_specs=[
            pl.BlockSpec(memory_space=pltpu.HBM),
            pl.BlockSpec(memory_space=pltpu.VMEM),
            pl.BlockSpec(memory_space=pltpu.VMEM),
        ],
        out_specs=pl.BlockSpec(memory_space=pltpu.HBM),
        out_shape=jax.ShapeDtypeStruct((b * hw, c), x.dtype),
        scratch_shapes=[
            pltpu.VMEM((K, R, c), jnp.float32),
            pltpu.VMEM((K, R, c), jnp.float32),
            pltpu.VMEM((hw, c), jnp.float32),
            pltpu.SemaphoreType.DMA((K,)),
            pltpu.SemaphoreType.DMA((K,)),
        ],
    )(xt, row_embed, col_embed)
    return out.reshape(b, h, w, c).transpose(0, 3, 1, 2)
